# baseline (speedup 1.0000x reference)
import jax
import jax.numpy as jnp
from jax.experimental import pallas as pl
from jax.experimental.pallas import tpu as pltpu

_D = 32
_V = 8192
_TB = 256
_VC = 1024
_NC = _V // _VC
_HALF = _NC // 2


def _argmin_kernel(x_ref, w_ref, il2_ref, wl2_ref, idx_ref,
                   bv_ref, bi_ref, ra_v_ref, ra_i_ref):
    c = pl.program_id(1)
    x = x_ref[...]
    w_c = w_ref[...]
    scores = jax.lax.dot_general(
        x, w_c, (((1,), (1,)), ((), ())),
        preferred_element_type=jnp.float32)
    neg = -(il2_ref[...] - 2.0 * scores + wl2_ref[...])
    m = jnp.max(neg, axis=1, keepdims=True)
    iot = jax.lax.broadcasted_iota(jnp.int32, (_TB, _VC), 1)
    i = jnp.min(jnp.where(neg == m, iot, _V),
                axis=1, keepdims=True) + c * _VC

    @pl.when((c == 0) | (c == _HALF))
    def _half_start():
        bv_ref[...] = m
        bi_ref[...] = i

    @pl.when((c != 0) & (c != _HALF))
    def _half_update():
        upd = m > bv_ref[...]
        bv_ref[...] = jnp.where(upd, m, bv_ref[...])
        bi_ref[...] = jnp.where(upd, i, bi_ref[...])

    @pl.when(c == _HALF - 1)
    def _round_half_a():
        ra_v_ref[...] = bv_ref[...].astype(jnp.bfloat16).astype(jnp.float32)
        ra_i_ref[...] = bi_ref[...]

    @pl.when(c == _NC - 1)
    def _emit():
        take_b = bv_ref[...] > ra_v_ref[...]
        idx_ref[...] = jnp.where(take_b, bi_ref[...], ra_i_ref[...])[None]


def _gather_kernel(idx_ref, x_ref, w_ref, q_ref, diff_ref):
    i = pl.program_id(0)
    c = pl.program_id(1)
    local = idx_ref[0, :, :] - c * _VC
    onehot = (jax.lax.broadcasted_iota(jnp.int32, (_TB, _VC), 1)
              == local).astype(jnp.float32)
    part = jax.lax.dot_general(
        onehot, w_ref[...], (((1,), (0,)), ((), ())),
        preferred_element_type=jnp.float32,
        precision=jax.lax.Precision.HIGHEST)

    @pl.when(c == 0)
    def _init():
        q_ref[...] = jnp.zeros((_TB, _D), jnp.float32)

    q_ref[...] += part

    @pl.when(c == _NC - 1)
    def _finish():
        x = x_ref[...]
        q = q_ref[...]
        d = x - q
        q_ref[...] = (q - x) + x
        blk = jnp.sum(d * d, axis=(0, 1), keepdims=True).reshape(1, 1)

        @pl.when(i == 0)
        def _zero():
            diff_ref[...] = jnp.zeros((1, 1), jnp.float32)

        diff_ref[...] += blk


def kernel(x, weight):
    n_tok = x.shape[0] * x.shape[1]
    flat = x.reshape(n_tok, _D).astype(jnp.float32)
    grid_i = n_tok // _TB

    inputs_l2 = jnp.sum(flat ** 2, axis=1, keepdims=True)
    codebook_l2 = jnp.sum(weight ** 2, axis=1, keepdims=True).T

    idx3 = pl.pallas_call(
        _argmin_kernel,
        grid=(grid_i, _NC),
        in_specs=[
            pl.BlockSpec((_TB, _D), lambda i, c: (i, 0)),
            pl.BlockSpec((_VC, _D), lambda i, c: (c, 0)),
            pl.BlockSpec((_TB, 1), lambda i, c: (i, 0)),
            pl.BlockSpec((1, _VC), lambda i, c: (0, c)),
        ],
        out_specs=pl.BlockSpec((1, _TB, 1), lambda i, c: (i, 0, 0)),
        out_shape=jax.ShapeDtypeStruct((grid_i, _TB, 1), jnp.int32),
        scratch_shapes=[
            pltpu.VMEM((_TB, 1), jnp.float32),
            pltpu.VMEM((_TB, 1), jnp.int32),
            pltpu.VMEM((_TB, 1), jnp.float32),
            pltpu.VMEM((_TB, 1), jnp.int32),
        ],
    )(flat, weight, inputs_l2, codebook_l2)

    quant, diff = pl.pallas_call(
        _gather_kernel,
        grid=(grid_i, _NC),
        in_specs=[
            pl.BlockSpec((1, _TB, 1), lambda i, c: (i, 0, 0)),
            pl.BlockSpec((_TB, _D), lambda i, c: (i, 0)),
            pl.BlockSpec((_VC, _D), lambda i, c: (c, 0)),
        ],
        out_specs=[
            pl.BlockSpec((_TB, _D), lambda i, c: (i, 0)),
            pl.BlockSpec((1, 1), lambda i, c: (0, 0)),
        ],
        out_shape=[
            jax.ShapeDtypeStruct((n_tok, _D), jnp.float32),
            jax.ShapeDtypeStruct((1, 1), jnp.float32),
        ],
    )(idx3, flat, weight)

    quant_feat_st = quant.reshape(x.shape)
    encoding_indices = idx3.reshape(x.shape[:2])
    quant_diff = (diff[0, 0] / jnp.float32(n_tok * _D)).astype(jnp.float32)
    return (quant_feat_st, encoding_indices, quant_diff)
